# baseline (speedup 1.0000x reference)
import functools

import jax
import jax.numpy as jnp
import numpy as np
from jax import lax
from jax.experimental import pallas as pl
from jax.experimental.pallas import tpu as pltpu
from jax.experimental.pallas import tpu_sc as plsc

N_POINTS = 65536
N_ENC = 4
N_LEVELS = 16
F_PER_LEVEL = 2
LOG2_T = 19
T = 1 << LOG2_T
BASE_RES = 16
PER_LEVEL_SCALE = 1.4472692012786865
P1_I32 = int(np.uint32(2654435761).view(np.int32))
P2_I32 = int(np.uint32(805459861).view(np.int32))

OUT_F = N_LEVELS * F_PER_LEVEL
ROW_F = N_ENC * F_PER_LEVEL
NROWS = N_LEVELS * T

NW = 32
PT = N_POINTS // NW
P = 512
G = P // 16
CH = PT // P
NI = P * 8

SPT = NROWS // NW
RC = 2048
RB = RC // 128
NCH = SPT // RC

_LEVELS = []
for _l in range(N_LEVELS):
    _scale = BASE_RES * (PER_LEVEL_SCALE ** _l) - 1.0
    _res = int(np.ceil(_scale)) + 1
    _LEVELS.append((np.float32(_scale), _res, (_res ** 3) <= T))


def _full(v, dtype=jnp.int32):
    return jnp.full((16,), v, dtype)


def _lane():
    return lax.iota(jnp.int32, 16)


def _relayout_sc(phys, table_t, in_a, in_b, out_a, out_b,
                 isem_a, isem_b, osem_a, osem_b):
    wid = lax.axis_index("s") * 2 + lax.axis_index("c")
    lvl = wid >> 1
    t_base = (wid & 1) * SPT
    iota8 = _lane() * 8

    def in_copies(tc, in_buf, isem):
        qb = lax.shift_right_logical(tc, 7)
        return [pltpu.make_async_copy(
            phys.at[pl.ds((e * N_LEVELS + lvl) * (T // 128) + qb, RB), :],
            in_buf.at[e], isem) for e in range(N_ENC)]

    def shuffle(in_buf, out_buf):
        def blk(b, carry):
            rows = [b * 128 + g * 16 + _lane() for g in range(8)]
            cols = [_full(j) for j in range(ROW_F)]
            for e in range(N_ENC):
                for f in range(F_PER_LEVEL):
                    for g in range(8):
                        v = in_buf[e, b, pl.ds(f * 128 + g * 16, 16)]
                        plsc.store_scatter(
                            out_buf, [rows[g], cols[e * 2 + f]], v)
            return carry
        lax.fori_loop(0, RB, blk, 0)

    def out_copy(tc, out_buf, osem):
        s0 = lvl * T + tc
        return pltpu.make_async_copy(
            out_buf, table_t.at[pl.ds(s0, RC), :], osem)

    bufs = ((in_a, isem_a, out_a, osem_a), (in_b, isem_b, out_b, osem_b))

    def pair_body(j, carry):
        for s, (in_buf, isem, out_buf, osem) in enumerate(bufs):
            ci = j * 2 + s
            tc = t_base + ci * RC
            @pl.when(j > 0)
            def _():
                out_copy(t_base, out_buf, osem).wait()
            for cp in in_copies(tc, in_buf, isem):
                cp.wait()
            shuffle(in_buf, out_buf)
            @pl.when(ci + 2 < NCH)
            def _():
                for cp in in_copies(tc + 2 * RC, in_buf, isem):
                    cp.start()
            out_copy(tc, out_buf, osem).start()
        return carry

    for s, (in_buf, isem, _, _) in enumerate(bufs):
        for cp in in_copies(t_base + s * RC, in_buf, isem):
            cp.start()
    lax.fori_loop(0, NCH // 2, pair_body, 0)
    for s, (_, _, out_buf, osem) in enumerate(bufs):
        out_copy(t_base, out_buf, osem).wait()


def _idx_pass(l, coords_v, idx_ref):
    scale, res, dense = _LEVELS[l]
    lbase = l * T

    def body(g, carry):
        pos_i = []
        for d in range(3):
            x = coords_v[d, pl.ds(g * 16, 16)]
            px = x * scale + np.float32(0.5)
            pos_i.append(px.astype(jnp.int32))
        if dense:
            xs = (pos_i[0], pos_i[0] + 1)
            ys = (pos_i[1] * res, (pos_i[1] + 1) * res)
            zs = (pos_i[2] * (res * res), (pos_i[2] + 1) * (res * res))
        else:
            xs = (pos_i[0], pos_i[0] + 1)
            ys = (pos_i[1] * P1_I32, pos_i[1] * P1_I32 + P1_I32)
            zs = (pos_i[2] * P2_I32, pos_i[2] * P2_I32 + P2_I32)
        base = g * 128
        for c in range(8):
            ox, oy, oz = c & 1, (c >> 1) & 1, (c >> 2) & 1
            if dense:
                h = xs[ox] + ys[oy] + zs[oz]
            else:
                h = (xs[ox] ^ ys[oy]) ^ zs[oz]
            idx_ref[pl.ds(base + c * 16, 16)] = (h & (T - 1)) + lbase
        return carry

    lax.fori_loop(0, G, body, 0)


def _blend_pass(l, coords_v, code_v, dst_ref, out_v):
    scale, _, _ = _LEVELS[l]
    j0, j1 = 2 * l, 2 * l + 1

    def body(g, carry):
        frac = []
        for d in range(3):
            x = coords_v[d, pl.ds(g * 16, 16)]
            px = x * scale + np.float32(0.5)
            pi = px.astype(jnp.int32)
            frac.append(px - pi.astype(jnp.float32))
        one = np.float32(1.0)
        wx = (one - frac[0], frac[0])
        wy = (one - frac[1], frac[1])
        wz = (one - frac[2], frac[2])
        wxy = tuple(wx[ox] * wy[oy] for oy in range(2) for ox in range(2))
        code = [code_v[e, pl.ds(g * 16, 16)] for e in range(N_ENC)]
        acc0 = jnp.zeros((16,), jnp.float32)
        acc1 = jnp.zeros((16,), jnp.float32)
        base = g * 128
        for c in range(8):
            ox, oy, oz = c & 1, (c >> 1) & 1, (c >> 2) & 1
            rb = base + c * 16 + _lane()
            r = [plsc.load_gather(dst_ref, [rb, _full(col)])
                 for col in range(ROW_F)]
            b0 = (code[0] * r[0] + code[1] * r[2]) + \
                 (code[2] * r[4] + code[3] * r[6])
            b1 = (code[0] * r[1] + code[1] * r[3]) + \
                 (code[2] * r[5] + code[3] * r[7])
            w = wxy[oy * 2 + ox] * wz[oz]
            acc0 = acc0 + w * b0
            acc1 = acc1 + w * b1
        cb = lax.shift_right_logical(g, 3)
        ug = (g & 7) * 16
        out_v[j0 >> 3, cb, j0 & 7, pl.ds(ug, 16)] = acc0
        out_v[j1 >> 3, cb, j1 & 7, pl.ds(ug, 16)] = acc1
        return carry

    lax.fori_loop(0, G, body, 0)


def _gather_sc(xyz, code, table_t, out4,
               coords_v, code_v, idx_a, idx_b, dst_a, dst_b, out_v,
               sem_a, sem_b):
    wid = lax.axis_index("s") * 2 + lax.axis_index("c")
    idx_bufs = (idx_a, idx_b)
    dst_bufs = (dst_a, dst_b)
    sems = (sem_a, sem_b)

    def chunk_body(ch, carry):
        base = wid * PT + ch * P
        for d in range(3):
            pltpu.sync_copy(xyz.at[pl.ds(d * N_POINTS + base, P)],
                            coords_v.at[d])
        for e in range(N_ENC):
            pltpu.sync_copy(code.at[pl.ds(e * N_POINTS + base, P)],
                            code_v.at[e])
        _idx_pass(0, coords_v, idx_bufs[0])
        pltpu.make_async_copy(table_t.at[idx_bufs[0]], dst_bufs[0],
                              sems[0]).start()
        for l in range(N_LEVELS):
            cur = l & 1
            nxt = 1 - cur
            if l + 1 < N_LEVELS:
                _idx_pass(l + 1, coords_v, idx_bufs[nxt])
                pltpu.make_async_copy(table_t.at[idx_bufs[nxt]],
                                      dst_bufs[nxt], sems[nxt]).start()
            pltpu.make_async_copy(table_t.at[idx_bufs[cur]], dst_bufs[cur],
                                  sems[cur]).wait()
            _blend_pass(l, coords_v, code_v, dst_bufs[cur], out_v)
        cb0 = lax.shift_right_logical(base, 7)
        for r in range(4):
            pltpu.sync_copy(out_v.at[r],
                            out4.at[r, pl.ds(cb0, P // 128), :, :])
        return carry

    lax.fori_loop(0, CH, chunk_body, 0)


@functools.cache
def _build_relayout():
    return pl.kernel(
        _relayout_sc,
        out_type=jax.ShapeDtypeStruct((NROWS, ROW_F), jnp.float32),
        mesh=plsc.VectorSubcoreMesh(core_axis_name="c", subcore_axis_name="s"),
        compiler_params=pltpu.CompilerParams(
            needs_layout_passes=False, use_tc_tiling_on_sc=False),
        scratch_types=[
            pltpu.VMEM((N_ENC, RB, 256), jnp.float32),
            pltpu.VMEM((N_ENC, RB, 256), jnp.float32),
            pltpu.VMEM((RC, ROW_F), jnp.float32),
            pltpu.VMEM((RC, ROW_F), jnp.float32),
            pltpu.SemaphoreType.DMA,
            pltpu.SemaphoreType.DMA,
            pltpu.SemaphoreType.DMA,
            pltpu.SemaphoreType.DMA,
        ],
    )


@functools.cache
def _build_gather():
    return pl.kernel(
        _gather_sc,
        out_type=jax.ShapeDtypeStruct((4, N_POINTS // 128, 8, 128),
                                      jnp.float32),
        mesh=plsc.VectorSubcoreMesh(core_axis_name="c", subcore_axis_name="s"),
        compiler_params=pltpu.CompilerParams(
            needs_layout_passes=False, use_tc_tiling_on_sc=False),
        scratch_types=[
            pltpu.VMEM((3, P), jnp.float32),
            pltpu.VMEM((N_ENC, P), jnp.float32),
            pltpu.VMEM((NI,), jnp.int32),
            pltpu.VMEM((NI,), jnp.int32),
            pltpu.VMEM((NI, ROW_F), jnp.float32),
            pltpu.VMEM((NI, ROW_F), jnp.float32),
            pltpu.VMEM((4, P // 128, 8, 128), jnp.float32),
            pltpu.SemaphoreType.DMA,
            pltpu.SemaphoreType.DMA,
        ],
    )


def kernel(in_tensor, conditioning_code, tables):
    phys = tables.reshape(N_ENC, N_LEVELS, T // 128, 128, F_PER_LEVEL)
    phys = phys.transpose(0, 1, 2, 4, 3)
    phys = phys.reshape(N_ENC * N_LEVELS * (T // 128), 256)
    xyz = in_tensor.T.reshape(-1)
    code = conditioning_code.T.reshape(-1)
    table_t = _build_relayout()(phys)
    out4 = _build_gather()(xyz, code, table_t)
    return out4.transpose(1, 3, 0, 2).reshape(N_POINTS, OUT_F)


# speedup vs baseline: 1.1611x; 1.1611x over previous
import functools

import jax
import jax.numpy as jnp
import numpy as np
from jax import lax
from jax.experimental import pallas as pl
from jax.experimental.pallas import tpu as pltpu
from jax.experimental.pallas import tpu_sc as plsc

N_POINTS = 65536
N_ENC = 4
N_LEVELS = 16
F_PER_LEVEL = 2
LOG2_T = 19
T = 1 << LOG2_T
BASE_RES = 16
PER_LEVEL_SCALE = 1.4472692012786865
P1_I32 = int(np.uint32(2654435761).view(np.int32))
P2_I32 = int(np.uint32(805459861).view(np.int32))

OUT_F = N_LEVELS * F_PER_LEVEL
ROW_F = N_ENC * F_PER_LEVEL
NROWS = N_LEVELS * T

NW = 32
PT = N_POINTS // NW
P = 512
G = P // 16
CH = PT // P
NI = P * 8

SPT = NROWS // NW
RC = 2048
RB = RC // 128
NCH = SPT // RC

_LEVELS = []
for _l in range(N_LEVELS):
    _scale = BASE_RES * (PER_LEVEL_SCALE ** _l) - 1.0
    _res = int(np.ceil(_scale)) + 1
    _LEVELS.append((np.float32(_scale), _res, (_res ** 3) <= T))


def _full(v, dtype=jnp.int32):
    return jnp.full((16,), v, dtype)


def _lane():
    return lax.iota(jnp.int32, 16)


def _relayout_sc(phys, table_t, in_a, in_b, out_a, out_b,
                 isem_a, isem_b, osem_a, osem_b):
    wid = lax.axis_index("s") * 2 + lax.axis_index("c")
    lvl = wid >> 1
    t_base = (wid & 1) * SPT
    iota8 = _lane() * 8

    def in_copies(tc, in_buf, isem):
        qb = lax.shift_right_logical(tc, 7)
        return [pltpu.make_async_copy(
            phys.at[pl.ds((e * N_LEVELS + lvl) * (T // 128) + qb, RB), :],
            in_buf.at[e], isem) for e in range(N_ENC)]

    def shuffle(in_buf, out_buf):
        def blk(b, carry):
            rows = [b * 128 + g * 16 + _lane() for g in range(8)]
            cols = [_full(j) for j in range(ROW_F)]
            for e in range(N_ENC):
                for f in range(F_PER_LEVEL):
                    vs = [in_buf[e, b, pl.ds(f * 128 + g * 16, 16)]
                          for g in range(8)]
                    for g in range(8):
                        plsc.store_scatter(
                            out_buf, [rows[g], cols[e * 2 + f]], vs[g])
            return carry
        lax.fori_loop(0, RB, blk, 0)

    def out_copy(tc, out_buf, osem):
        s0 = lvl * T + tc
        return pltpu.make_async_copy(
            out_buf, table_t.at[pl.ds(s0, RC), :], osem)

    bufs = ((in_a, isem_a, out_a, osem_a), (in_b, isem_b, out_b, osem_b))

    def pair_body(j, carry):
        for s, (in_buf, isem, out_buf, osem) in enumerate(bufs):
            ci = j * 2 + s
            tc = t_base + ci * RC
            @pl.when(j > 0)
            def _():
                out_copy(t_base, out_buf, osem).wait()
            for cp in in_copies(tc, in_buf, isem):
                cp.wait()
            shuffle(in_buf, out_buf)
            @pl.when(ci + 2 < NCH)
            def _():
                for cp in in_copies(tc + 2 * RC, in_buf, isem):
                    cp.start()
            out_copy(tc, out_buf, osem).start()
        return carry

    for s, (in_buf, isem, _, _) in enumerate(bufs):
        for cp in in_copies(t_base + s * RC, in_buf, isem):
            cp.start()
    lax.fori_loop(0, NCH // 2, pair_body, 0)
    for s, (_, _, out_buf, osem) in enumerate(bufs):
        out_copy(t_base, out_buf, osem).wait()


def _idx_pass(l, coords_v, idx_ref):
    scale, res, dense = _LEVELS[l]
    lbase = l * T

    def body(g, carry):
        pos_i = []
        for d in range(3):
            x = coords_v[d, pl.ds(g * 16, 16)]
            px = x * scale + np.float32(0.5)
            pos_i.append(px.astype(jnp.int32))
        if dense:
            xs = (pos_i[0], pos_i[0] + 1)
            ys = (pos_i[1] * res, (pos_i[1] + 1) * res)
            zs = (pos_i[2] * (res * res), (pos_i[2] + 1) * (res * res))
        else:
            xs = (pos_i[0], pos_i[0] + 1)
            ys = (pos_i[1] * P1_I32, pos_i[1] * P1_I32 + P1_I32)
            zs = (pos_i[2] * P2_I32, pos_i[2] * P2_I32 + P2_I32)
        base = g * 128
        for c in range(8):
            ox, oy, oz = c & 1, (c >> 1) & 1, (c >> 2) & 1
            if dense:
                h = xs[ox] + ys[oy] + zs[oz]
            else:
                h = (xs[ox] ^ ys[oy]) ^ zs[oz]
            idx_ref[pl.ds(base + c * 16, 16)] = (h & (T - 1)) + lbase
        return carry

    lax.fori_loop(0, G, body, 0)


def _blend_pass(l, coords_v, code_v, dst_ref, out_v):
    scale, _, _ = _LEVELS[l]
    j0, j1 = 2 * l, 2 * l + 1

    def body(g, carry):
        frac = []
        for d in range(3):
            x = coords_v[d, pl.ds(g * 16, 16)]
            px = x * scale + np.float32(0.5)
            pi = px.astype(jnp.int32)
            frac.append(px - pi.astype(jnp.float32))
        one = np.float32(1.0)
        wx = (one - frac[0], frac[0])
        wy = (one - frac[1], frac[1])
        wz = (one - frac[2], frac[2])
        wxy = tuple(wx[ox] * wy[oy] for oy in range(2) for ox in range(2))
        code = [code_v[e, pl.ds(g * 16, 16)] for e in range(N_ENC)]
        acc0 = jnp.zeros((16,), jnp.float32)
        acc1 = jnp.zeros((16,), jnp.float32)
        base = g * 128
        for c in range(8):
            ox, oy, oz = c & 1, (c >> 1) & 1, (c >> 2) & 1
            rb = base + c * 16 + _lane()
            r = [plsc.load_gather(dst_ref, [rb, _full(col)])
                 for col in range(ROW_F)]
            b0 = (code[0] * r[0] + code[1] * r[2]) + \
                 (code[2] * r[4] + code[3] * r[6])
            b1 = (code[0] * r[1] + code[1] * r[3]) + \
                 (code[2] * r[5] + code[3] * r[7])
            w = wxy[oy * 2 + ox] * wz[oz]
            acc0 = acc0 + w * b0
            acc1 = acc1 + w * b1
        cb = lax.shift_right_logical(g, 3)
        ug = (g & 7) * 16
        out_v[j0 >> 3, cb, j0 & 7, pl.ds(ug, 16)] = acc0
        out_v[j1 >> 3, cb, j1 & 7, pl.ds(ug, 16)] = acc1
        return carry

    lax.fori_loop(0, G, body, 0)


def _gather_sc(xyz, code, table_t, out4,
               coords_v, code_v, idx_a, idx_b, dst_a, dst_b, out_v,
               sem_a, sem_b):
    wid = lax.axis_index("s") * 2 + lax.axis_index("c")
    idx_bufs = (idx_a, idx_b)
    dst_bufs = (dst_a, dst_b)
    sems = (sem_a, sem_b)

    def chunk_body(ch, carry):
        base = wid * PT + ch * P
        for d in range(3):
            pltpu.sync_copy(xyz.at[pl.ds(d * N_POINTS + base, P)],
                            coords_v.at[d])
        for e in range(N_ENC):
            pltpu.sync_copy(code.at[pl.ds(e * N_POINTS + base, P)],
                            code_v.at[e])
        _idx_pass(0, coords_v, idx_bufs[0])
        pltpu.make_async_copy(table_t.at[idx_bufs[0]], dst_bufs[0],
                              sems[0]).start()
        for l in range(N_LEVELS):
            cur = l & 1
            nxt = 1 - cur
            if l + 1 < N_LEVELS:
                _idx_pass(l + 1, coords_v, idx_bufs[nxt])
                pltpu.make_async_copy(table_t.at[idx_bufs[nxt]],
                                      dst_bufs[nxt], sems[nxt]).start()
            pltpu.make_async_copy(table_t.at[idx_bufs[cur]], dst_bufs[cur],
                                  sems[cur]).wait()
            _blend_pass(l, coords_v, code_v, dst_bufs[cur], out_v)
        cb0 = lax.shift_right_logical(base, 7)
        for r in range(4):
            pltpu.sync_copy(out_v.at[r],
                            out4.at[r, pl.ds(cb0, P // 128), :, :])
        return carry

    lax.fori_loop(0, CH, chunk_body, 0)


@functools.cache
def _build_relayout():
    return pl.kernel(
        _relayout_sc,
        out_type=jax.ShapeDtypeStruct((NROWS, ROW_F), jnp.float32),
        mesh=plsc.VectorSubcoreMesh(core_axis_name="c", subcore_axis_name="s"),
        compiler_params=pltpu.CompilerParams(
            needs_layout_passes=False, use_tc_tiling_on_sc=False),
        scratch_types=[
            pltpu.VMEM((N_ENC, RB, 256), jnp.float32),
            pltpu.VMEM((N_ENC, RB, 256), jnp.float32),
            pltpu.VMEM((RC, ROW_F), jnp.float32),
            pltpu.VMEM((RC, ROW_F), jnp.float32),
            pltpu.SemaphoreType.DMA,
            pltpu.SemaphoreType.DMA,
            pltpu.SemaphoreType.DMA,
            pltpu.SemaphoreType.DMA,
        ],
    )


@functools.cache
def _build_gather():
    return pl.kernel(
        _gather_sc,
        out_type=jax.ShapeDtypeStruct((4, N_POINTS // 128, 8, 128),
                                      jnp.float32),
        mesh=plsc.VectorSubcoreMesh(core_axis_name="c", subcore_axis_name="s"),
        compiler_params=pltpu.CompilerParams(
            needs_layout_passes=False, use_tc_tiling_on_sc=False),
        scratch_types=[
            pltpu.VMEM((3, P), jnp.float32),
            pltpu.VMEM((N_ENC, P), jnp.float32),
            pltpu.VMEM((NI,), jnp.int32),
            pltpu.VMEM((NI,), jnp.int32),
            pltpu.VMEM((NI, ROW_F), jnp.float32),
            pltpu.VMEM((NI, ROW_F), jnp.float32),
            pltpu.VMEM((4, P // 128, 8, 128), jnp.float32),
            pltpu.SemaphoreType.DMA,
            pltpu.SemaphoreType.DMA,
        ],
    )


def kernel(in_tensor, conditioning_code, tables):
    phys = tables.reshape(N_ENC, N_LEVELS, T // 128, 128, F_PER_LEVEL)
    phys = phys.transpose(0, 1, 2, 4, 3)
    phys = phys.reshape(N_ENC * N_LEVELS * (T // 128), 256)
    xyz = in_tensor.T.reshape(-1)
    code = conditioning_code.T.reshape(-1)
    table_t = _build_relayout()(phys)
    out4 = _build_gather()(xyz, code, table_t)
    return out4.transpose(1, 3, 0, 2).reshape(N_POINTS, OUT_F)


# speedup vs baseline: 1.3023x; 1.1216x over previous
import functools

import jax
import jax.numpy as jnp
import numpy as np
from jax import lax
from jax.experimental import pallas as pl
from jax.experimental.pallas import tpu as pltpu
from jax.experimental.pallas import tpu_sc as plsc

N_POINTS = 65536
N_ENC = 4
N_LEVELS = 16
F_PER_LEVEL = 2
LOG2_T = 19
T = 1 << LOG2_T
BASE_RES = 16
PER_LEVEL_SCALE = 1.4472692012786865
P1_I32 = int(np.uint32(2654435761).view(np.int32))
P2_I32 = int(np.uint32(805459861).view(np.int32))

OUT_F = N_LEVELS * F_PER_LEVEL
ROW_F = N_ENC * F_PER_LEVEL
NROWS = N_LEVELS * T

NW = 32
PT = N_POINTS // NW
P = 512
G = P // 16
CH = PT // P
NI = P * 8

RC = 2048
RB = RC // 128

_LEVELS = []
for _l in range(N_LEVELS):
    _scale = BASE_RES * (PER_LEVEL_SCALE ** _l) - 1.0
    _res = int(np.ceil(_scale)) + 1
    _LEVELS.append((np.float32(_scale), _res, (_res ** 3) <= T))

_NCH_L = []
for _scale, _res, _dense in _LEVELS:
    _active = min(_res * (1 + _res + _res * _res) + 1, T) if _dense else T
    _NCH_L.append(-(-_active // RC))
_CUM = [int(c) for c in np.cumsum(_NCH_L)]
NCHT = _CUM[-1]
_MAXC = -(-NCHT // NW)
NPAIRS = -(-_MAXC // 2)
_REM = NCHT - NW * (NCHT // NW)


def _full(v, dtype=jnp.int32):
    return jnp.full((16,), v, dtype)


def _lane():
    return lax.iota(jnp.int32, 16)


def _relayout_sc(phys, table_t, in_a, in_b, out_a, out_b,
                 isem_a, isem_b, osem_a, osem_b):
    wid = lax.axis_index("s") * 2 + lax.axis_index("c")
    my_n = jnp.where(wid < _REM, _MAXC, _MAXC - 1) if _REM else _MAXC
    my_last = my_n - 1

    def chunk_params(i):
        k = wid + NW * jnp.minimum(i, my_last)
        lvl = jnp.int32(0)
        start = jnp.int32(0)
        for c in _CUM[:-1]:
            ge = (k >= c).astype(jnp.int32)
            lvl = lvl + ge
            start = jnp.where(k >= c, jnp.int32(c), start)
        return lvl, (k - start) * RC

    def in_copies(lvl, tc, in_buf, isem):
        qb = lax.shift_right_logical(tc, 7)
        return [pltpu.make_async_copy(
            phys.at[pl.ds((e * N_LEVELS + lvl) * (T // 128) + qb, RB), :],
            in_buf.at[e], isem) for e in range(N_ENC)]

    def shuffle(in_buf, out_buf):
        def blk(b, carry):
            rows = [b * 128 + g * 16 + _lane() for g in range(8)]
            cols = [_full(j) for j in range(ROW_F)]
            for e in range(N_ENC):
                for f in range(F_PER_LEVEL):
                    vs = [in_buf[e, b, pl.ds(f * 128 + g * 16, 16)]
                          for g in range(8)]
                    for g in range(8):
                        plsc.store_scatter(
                            out_buf, [rows[g], cols[e * 2 + f]], vs[g])
            return carry
        lax.fori_loop(0, RB, blk, 0)

    def out_copy(lvl, tc, out_buf, osem):
        return pltpu.make_async_copy(
            out_buf, table_t.at[pl.ds(lvl * T + tc, RC), :], osem)

    bufs = ((in_a, isem_a, out_a, osem_a), (in_b, isem_b, out_b, osem_b))

    def pair_body(j, carry):
        for s, (in_buf, isem, out_buf, osem) in enumerate(bufs):
            i = j * 2 + s
            lvl, tc = chunk_params(i)
            @pl.when(j > 0)
            def _():
                out_copy(0, 0, out_buf, osem).wait()
            for cp in in_copies(0, 0, in_buf, isem):
                cp.wait()
            shuffle(in_buf, out_buf)
            @pl.when(i < 2 * NPAIRS - 2)
            def _():
                lvl2, tc2 = chunk_params(i + 2)
                for cp in in_copies(lvl2, tc2, in_buf, isem):
                    cp.start()
            out_copy(lvl, tc, out_buf, osem).start()
        return carry

    for s, (in_buf, isem, _, _) in enumerate(bufs):
        lvl0, tc0 = chunk_params(s)
        for cp in in_copies(lvl0, tc0, in_buf, isem):
            cp.start()
    lax.fori_loop(0, NPAIRS, pair_body, 0)
    for s, (_, _, out_buf, osem) in enumerate(bufs):
        out_copy(0, 0, out_buf, osem).wait()


def _idx_pass(l, coords_v, idx_ref):
    scale, res, dense = _LEVELS[l]
    lbase = l * T

    def body(g, carry):
        pos_i = []
        for d in range(3):
            x = coords_v[d, pl.ds(g * 16, 16)]
            px = x * scale + np.float32(0.5)
            pos_i.append(px.astype(jnp.int32))
        if dense:
            xs = (pos_i[0], pos_i[0] + 1)
            ys = (pos_i[1] * res, (pos_i[1] + 1) * res)
            zs = (pos_i[2] * (res * res), (pos_i[2] + 1) * (res * res))
        else:
            xs = (pos_i[0], pos_i[0] + 1)
            ys = (pos_i[1] * P1_I32, pos_i[1] * P1_I32 + P1_I32)
            zs = (pos_i[2] * P2_I32, pos_i[2] * P2_I32 + P2_I32)
        base = g * 128
        for c in range(8):
            ox, oy, oz = c & 1, (c >> 1) & 1, (c >> 2) & 1
            if dense:
                h = xs[ox] + ys[oy] + zs[oz]
            else:
                h = (xs[ox] ^ ys[oy]) ^ zs[oz]
            idx_ref[pl.ds(base + c * 16, 16)] = (h & (T - 1)) + lbase
        return carry

    lax.fori_loop(0, G, body, 0)


def _blend_pass(l, coords_v, code_v, dst_ref, out_v):
    scale, _, _ = _LEVELS[l]
    j0, j1 = 2 * l, 2 * l + 1

    def body(g, carry):
        frac = []
        for d in range(3):
            x = coords_v[d, pl.ds(g * 16, 16)]
            px = x * scale + np.float32(0.5)
            pi = px.astype(jnp.int32)
            frac.append(px - pi.astype(jnp.float32))
        one = np.float32(1.0)
        wx = (one - frac[0], frac[0])
        wy = (one - frac[1], frac[1])
        wz = (one - frac[2], frac[2])
        wxy = tuple(wx[ox] * wy[oy] for oy in range(2) for ox in range(2))
        code = [code_v[e, pl.ds(g * 16, 16)] for e in range(N_ENC)]
        acc0 = jnp.zeros((16,), jnp.float32)
        acc1 = jnp.zeros((16,), jnp.float32)
        base = g * 128
        for c in range(8):
            ox, oy, oz = c & 1, (c >> 1) & 1, (c >> 2) & 1
            rb = base + c * 16 + _lane()
            r = [plsc.load_gather(dst_ref, [rb, _full(col)])
                 for col in range(ROW_F)]
            b0 = (code[0] * r[0] + code[1] * r[2]) + \
                 (code[2] * r[4] + code[3] * r[6])
            b1 = (code[0] * r[1] + code[1] * r[3]) + \
                 (code[2] * r[5] + code[3] * r[7])
            w = wxy[oy * 2 + ox] * wz[oz]
            acc0 = acc0 + w * b0
            acc1 = acc1 + w * b1
        cb = lax.shift_right_logical(g, 3)
        ug = (g & 7) * 16
        out_v[j0 >> 3, cb, j0 & 7, pl.ds(ug, 16)] = acc0
        out_v[j1 >> 3, cb, j1 & 7, pl.ds(ug, 16)] = acc1
        return carry

    lax.fori_loop(0, G, body, 0)


def _gather_sc(xyz, code, table_t, out4,
               coords_v, code_v, idx_a, idx_b, dst_a, dst_b, out_v,
               sem_a, sem_b):
    wid = lax.axis_index("s") * 2 + lax.axis_index("c")
    idx_bufs = (idx_a, idx_b)
    dst_bufs = (dst_a, dst_b)
    sems = (sem_a, sem_b)

    def chunk_body(ch, carry):
        base = wid * PT + ch * P
        for d in range(3):
            pltpu.sync_copy(xyz.at[pl.ds(d * N_POINTS + base, P)],
                            coords_v.at[d])
        for e in range(N_ENC):
            pltpu.sync_copy(code.at[pl.ds(e * N_POINTS + base, P)],
                            code_v.at[e])
        _idx_pass(0, coords_v, idx_bufs[0])
        pltpu.make_async_copy(table_t.at[idx_bufs[0]], dst_bufs[0],
                              sems[0]).start()
        for l in range(N_LEVELS):
            cur = l & 1
            nxt = 1 - cur
            if l + 1 < N_LEVELS:
                _idx_pass(l + 1, coords_v, idx_bufs[nxt])
                pltpu.make_async_copy(table_t.at[idx_bufs[nxt]],
                                      dst_bufs[nxt], sems[nxt]).start()
            pltpu.make_async_copy(table_t.at[idx_bufs[cur]], dst_bufs[cur],
                                  sems[cur]).wait()
            _blend_pass(l, coords_v, code_v, dst_bufs[cur], out_v)
        cb0 = lax.shift_right_logical(base, 7)
        for r in range(4):
            pltpu.sync_copy(out_v.at[r],
                            out4.at[r, pl.ds(cb0, P // 128), :, :])
        return carry

    lax.fori_loop(0, CH, chunk_body, 0)


@functools.cache
def _build_relayout():
    return pl.kernel(
        _relayout_sc,
        out_type=jax.ShapeDtypeStruct((NROWS, ROW_F), jnp.float32),
        mesh=plsc.VectorSubcoreMesh(core_axis_name="c", subcore_axis_name="s"),
        compiler_params=pltpu.CompilerParams(
            needs_layout_passes=False, use_tc_tiling_on_sc=False),
        scratch_types=[
            pltpu.VMEM((N_ENC, RB, 256), jnp.float32),
            pltpu.VMEM((N_ENC, RB, 256), jnp.float32),
            pltpu.VMEM((RC, ROW_F), jnp.float32),
            pltpu.VMEM((RC, ROW_F), jnp.float32),
            pltpu.SemaphoreType.DMA,
            pltpu.SemaphoreType.DMA,
            pltpu.SemaphoreType.DMA,
            pltpu.SemaphoreType.DMA,
        ],
    )


@functools.cache
def _build_gather():
    return pl.kernel(
        _gather_sc,
        out_type=jax.ShapeDtypeStruct((4, N_POINTS // 128, 8, 128),
                                      jnp.float32),
        mesh=plsc.VectorSubcoreMesh(core_axis_name="c", subcore_axis_name="s"),
        compiler_params=pltpu.CompilerParams(
            needs_layout_passes=False, use_tc_tiling_on_sc=False),
        scratch_types=[
            pltpu.VMEM((3, P), jnp.float32),
            pltpu.VMEM((N_ENC, P), jnp.float32),
            pltpu.VMEM((NI,), jnp.int32),
            pltpu.VMEM((NI,), jnp.int32),
            pltpu.VMEM((NI, ROW_F), jnp.float32),
            pltpu.VMEM((NI, ROW_F), jnp.float32),
            pltpu.VMEM((4, P // 128, 8, 128), jnp.float32),
            pltpu.SemaphoreType.DMA,
            pltpu.SemaphoreType.DMA,
        ],
    )


def kernel(in_tensor, conditioning_code, tables):
    phys = tables.reshape(N_ENC, N_LEVELS, T // 128, 128, F_PER_LEVEL)
    phys = phys.transpose(0, 1, 2, 4, 3)
    phys = phys.reshape(N_ENC * N_LEVELS * (T // 128), 256)
    xyz = in_tensor.T.reshape(-1)
    code = conditioning_code.T.reshape(-1)
    table_t = _build_relayout()(phys)
    out4 = _build_gather()(xyz, code, table_t)
    return out4.transpose(1, 3, 0, 2).reshape(N_POINTS, OUT_F)


# speedup vs baseline: 1.3682x; 1.0506x over previous
import functools

import jax
import jax.numpy as jnp
import numpy as np
from jax import lax
from jax.experimental import pallas as pl
from jax.experimental.pallas import tpu as pltpu
from jax.experimental.pallas import tpu_sc as plsc

N_POINTS = 65536
N_ENC = 4
N_LEVELS = 16
F_PER_LEVEL = 2
LOG2_T = 19
T = 1 << LOG2_T
BASE_RES = 16
PER_LEVEL_SCALE = 1.4472692012786865
P1_I32 = int(np.uint32(2654435761).view(np.int32))
P2_I32 = int(np.uint32(805459861).view(np.int32))

OUT_F = N_LEVELS * F_PER_LEVEL
ROW_F = N_ENC * F_PER_LEVEL
NROWS = N_LEVELS * T

NW = 32
PT = N_POINTS // NW
P = 512
G = P // 16
CH = PT // P
NI = P * 8

RC = 2048
RB = RC // 128

_LEVELS = []
for _l in range(N_LEVELS):
    _scale = BASE_RES * (PER_LEVEL_SCALE ** _l) - 1.0
    _res = int(np.ceil(_scale)) + 1
    _LEVELS.append((np.float32(_scale), _res, (_res ** 3) <= T))

_NCH_L = []
for _scale, _res, _dense in _LEVELS:
    _active = min(_res * (1 + _res + _res * _res) + 1, T) if _dense else T
    _NCH_L.append(-(-_active // RC))
_CUM = [int(c) for c in np.cumsum(_NCH_L)]
NCHT = _CUM[-1]
_MAXC = -(-NCHT // NW)
NPAIRS = -(-_MAXC // 2)
_REM = NCHT - NW * (NCHT // NW)

N_STAGED = 3
_STG_PAD = []
_STG_OFF = []
_off = 0
for _l in range(N_STAGED):
    _res = _LEVELS[_l][1]
    _a = _res * (1 + _res + _res * _res) + 1
    _pad = -(-_a // 128) * 128
    _STG_OFF.append(_off)
    _STG_PAD.append(_pad)
    _off += _pad
STG_ROWS = _off
_STG_PER_TILE = [p // 16 for p in _STG_PAD]


def _full(v, dtype=jnp.int32):
    return jnp.full((16,), v, dtype)


def _lane():
    return lax.iota(jnp.int32, 16)


def _relayout_sc(phys, table_t, in_a, in_b, out_a, out_b,
                 isem_a, isem_b, osem_a, osem_b):
    wid = lax.axis_index("s") * 2 + lax.axis_index("c")
    my_n = jnp.where(wid < _REM, _MAXC, _MAXC - 1) if _REM else _MAXC
    my_last = my_n - 1

    def chunk_params(i):
        k = wid + NW * jnp.minimum(i, my_last)
        lvl = jnp.int32(0)
        start = jnp.int32(0)
        for c in _CUM[:-1]:
            ge = (k >= c).astype(jnp.int32)
            lvl = lvl + ge
            start = jnp.where(k >= c, jnp.int32(c), start)
        return lvl, (k - start) * RC

    def in_copies(lvl, tc, in_buf, isem):
        qb = lax.shift_right_logical(tc, 7)
        return [pltpu.make_async_copy(
            phys.at[pl.ds((e * N_LEVELS + lvl) * (T // 128) + qb, RB), :],
            in_buf.at[e], isem) for e in range(N_ENC)]

    def shuffle(in_buf, out_buf):
        def blk(b, carry):
            rows = [b * 128 + g * 16 + _lane() for g in range(8)]
            cols = [_full(j) for j in range(ROW_F)]
            for e in range(N_ENC):
                for f in range(F_PER_LEVEL):
                    vs = [in_buf[e, b, pl.ds(f * 128 + g * 16, 16)]
                          for g in range(8)]
                    for g in range(8):
                        plsc.store_scatter(
                            out_buf, [rows[g], cols[e * 2 + f]], vs[g])
            return carry
        lax.fori_loop(0, RB, blk, 0)

    def out_copy(lvl, tc, out_buf, osem):
        return pltpu.make_async_copy(
            out_buf, table_t.at[pl.ds(lvl * T + tc, RC), :], osem)

    bufs = ((in_a, isem_a, out_a, osem_a), (in_b, isem_b, out_b, osem_b))

    def pair_body(j, carry):
        for s, (in_buf, isem, out_buf, osem) in enumerate(bufs):
            i = j * 2 + s
            lvl, tc = chunk_params(i)
            @pl.when(j > 0)
            def _():
                out_copy(0, 0, out_buf, osem).wait()
            for cp in in_copies(0, 0, in_buf, isem):
                cp.wait()
            shuffle(in_buf, out_buf)
            @pl.when(i < 2 * NPAIRS - 2)
            def _():
                lvl2, tc2 = chunk_params(i + 2)
                for cp in in_copies(lvl2, tc2, in_buf, isem):
                    cp.start()
            out_copy(lvl, tc, out_buf, osem).start()
        return carry

    for s, (in_buf, isem, _, _) in enumerate(bufs):
        lvl0, tc0 = chunk_params(s)
        for cp in in_copies(lvl0, tc0, in_buf, isem):
            cp.start()
    lax.fori_loop(0, NPAIRS, pair_body, 0)
    for s, (_, _, out_buf, osem) in enumerate(bufs):
        out_copy(0, 0, out_buf, osem).wait()


def _idx_pass(l, coords_v, idx_ref):
    scale, res, dense = _LEVELS[l]
    lbase = _STG_OFF[l] if l < N_STAGED else l * T

    def body(g, carry):
        pos_i = []
        for d in range(3):
            x = coords_v[d, pl.ds(g * 16, 16)]
            px = x * scale + np.float32(0.5)
            pos_i.append(px.astype(jnp.int32))
        if dense:
            xs = (pos_i[0], pos_i[0] + 1)
            ys = (pos_i[1] * res, (pos_i[1] + 1) * res)
            zs = (pos_i[2] * (res * res), (pos_i[2] + 1) * (res * res))
        else:
            xs = (pos_i[0], pos_i[0] + 1)
            ys = (pos_i[1] * P1_I32, pos_i[1] * P1_I32 + P1_I32)
            zs = (pos_i[2] * P2_I32, pos_i[2] * P2_I32 + P2_I32)
        base = g * 128
        for c in range(8):
            ox, oy, oz = c & 1, (c >> 1) & 1, (c >> 2) & 1
            if dense:
                h = xs[ox] + ys[oy] + zs[oz]
            else:
                h = (xs[ox] ^ ys[oy]) ^ zs[oz]
            idx_ref[pl.ds(base + c * 16, 16)] = (h & (T - 1)) + lbase
        return carry

    lax.fori_loop(0, G, body, 0)


def _blend_pass(l, coords_v, code_v, dst_ref, out_v):
    scale, _, _ = _LEVELS[l]
    j0, j1 = 2 * l, 2 * l + 1

    def body(g, carry):
        frac = []
        for d in range(3):
            x = coords_v[d, pl.ds(g * 16, 16)]
            px = x * scale + np.float32(0.5)
            pi = px.astype(jnp.int32)
            frac.append(px - pi.astype(jnp.float32))
        one = np.float32(1.0)
        wx = (one - frac[0], frac[0])
        wy = (one - frac[1], frac[1])
        wz = (one - frac[2], frac[2])
        wxy = tuple(wx[ox] * wy[oy] for oy in range(2) for ox in range(2))
        code = [code_v[e, pl.ds(g * 16, 16)] for e in range(N_ENC)]
        acc0 = jnp.zeros((16,), jnp.float32)
        acc1 = jnp.zeros((16,), jnp.float32)
        base = g * 128
        for c in range(8):
            ox, oy, oz = c & 1, (c >> 1) & 1, (c >> 2) & 1
            rb = base + c * 16 + _lane()
            r = [plsc.load_gather(dst_ref, [rb, _full(col)])
                 for col in range(ROW_F)]
            b0 = (code[0] * r[0] + code[1] * r[2]) + \
                 (code[2] * r[4] + code[3] * r[6])
            b1 = (code[0] * r[1] + code[1] * r[3]) + \
                 (code[2] * r[5] + code[3] * r[7])
            w = wxy[oy * 2 + ox] * wz[oz]
            acc0 = acc0 + w * b0
            acc1 = acc1 + w * b1
        cb = lax.shift_right_logical(g, 3)
        ug = (g & 7) * 16
        out_v[j0 >> 3, cb, j0 & 7, pl.ds(ug, 16)] = acc0
        out_v[j1 >> 3, cb, j1 & 7, pl.ds(ug, 16)] = acc1
        return carry

    lax.fori_loop(0, G, body, 0)


def _gather_sc(xyz, code, table_t, out4,
               coords_v, code_v, idx_a, idx_b, dst_a, dst_b, out_v,
               stg_v, sem_a, sem_b, fsem):
    wid = lax.axis_index("s") * 2 + lax.axis_index("c")
    sid = lax.axis_index("s")
    idx_bufs = (idx_a, idx_b)
    dst_bufs = (dst_a, dst_b)
    sems = (sem_a, sem_b)

    fills = []
    for l in range(N_STAGED):
        n = _STG_PER_TILE[l]
        fills.append(pltpu.make_async_copy(
            table_t.at[pl.ds(l * T + sid * n, n), :],
            stg_v.at[pl.ds(_STG_OFF[l] + sid * n, n), :], fsem))
    for cp in fills:
        cp.start()
    for cp in fills:
        cp.wait()
    plsc.subcore_barrier()

    def src(l):
        return stg_v if l < N_STAGED else table_t

    def chunk_body(ch, carry):
        base = wid * PT + ch * P
        for d in range(3):
            pltpu.sync_copy(xyz.at[pl.ds(d * N_POINTS + base, P)],
                            coords_v.at[d])
        for e in range(N_ENC):
            pltpu.sync_copy(code.at[pl.ds(e * N_POINTS + base, P)],
                            code_v.at[e])
        _idx_pass(0, coords_v, idx_bufs[0])
        pltpu.make_async_copy(src(0).at[idx_bufs[0]], dst_bufs[0],
                              sems[0]).start()
        for l in range(N_LEVELS):
            cur = l & 1
            nxt = 1 - cur
            if l + 1 < N_LEVELS:
                _idx_pass(l + 1, coords_v, idx_bufs[nxt])
                pltpu.make_async_copy(src(l + 1).at[idx_bufs[nxt]],
                                      dst_bufs[nxt], sems[nxt]).start()
            pltpu.make_async_copy(src(l).at[idx_bufs[cur]], dst_bufs[cur],
                                  sems[cur]).wait()
            _blend_pass(l, coords_v, code_v, dst_bufs[cur], out_v)
        cb0 = lax.shift_right_logical(base, 7)
        for r in range(4):
            pltpu.sync_copy(out_v.at[r],
                            out4.at[r, pl.ds(cb0, P // 128), :, :])
        return carry

    lax.fori_loop(0, CH, chunk_body, 0)


@functools.cache
def _build_relayout():
    return pl.kernel(
        _relayout_sc,
        out_type=jax.ShapeDtypeStruct((NROWS, ROW_F), jnp.float32),
        mesh=plsc.VectorSubcoreMesh(core_axis_name="c", subcore_axis_name="s"),
        compiler_params=pltpu.CompilerParams(
            needs_layout_passes=False, use_tc_tiling_on_sc=False),
        scratch_types=[
            pltpu.VMEM((N_ENC, RB, 256), jnp.float32),
            pltpu.VMEM((N_ENC, RB, 256), jnp.float32),
            pltpu.VMEM((RC, ROW_F), jnp.float32),
            pltpu.VMEM((RC, ROW_F), jnp.float32),
            pltpu.SemaphoreType.DMA,
            pltpu.SemaphoreType.DMA,
            pltpu.SemaphoreType.DMA,
            pltpu.SemaphoreType.DMA,
        ],
    )


@functools.cache
def _build_gather():
    return pl.kernel(
        _gather_sc,
        out_type=jax.ShapeDtypeStruct((4, N_POINTS // 128, 8, 128),
                                      jnp.float32),
        mesh=plsc.VectorSubcoreMesh(core_axis_name="c", subcore_axis_name="s"),
        compiler_params=pltpu.CompilerParams(
            needs_layout_passes=False, use_tc_tiling_on_sc=False),
        scratch_types=[
            pltpu.VMEM((3, P), jnp.float32),
            pltpu.VMEM((N_ENC, P), jnp.float32),
            pltpu.VMEM((NI,), jnp.int32),
            pltpu.VMEM((NI,), jnp.int32),
            pltpu.VMEM((NI, ROW_F), jnp.float32),
            pltpu.VMEM((NI, ROW_F), jnp.float32),
            pltpu.VMEM((4, P // 128, 8, 128), jnp.float32),
            pltpu.VMEM_SHARED((STG_ROWS, ROW_F), jnp.float32),
            pltpu.SemaphoreType.DMA,
            pltpu.SemaphoreType.DMA,
            pltpu.SemaphoreType.DMA,
        ],
    )


def kernel(in_tensor, conditioning_code, tables):
    phys = tables.reshape(N_ENC, N_LEVELS, T // 128, 128, F_PER_LEVEL)
    phys = phys.transpose(0, 1, 2, 4, 3)
    phys = phys.reshape(N_ENC * N_LEVELS * (T // 128), 256)
    xyz = in_tensor.T.reshape(-1)
    code = conditioning_code.T.reshape(-1)
    table_t = _build_relayout()(phys)
    out4 = _build_gather()(xyz, code, table_t)
    return out4.transpose(1, 3, 0, 2).reshape(N_POINTS, OUT_F)
